# baseline (speedup 1.0000x reference)
import functools

import jax
import jax.numpy as jnp
from jax import lax
from jax.experimental import pallas as pl
from jax.experimental.pallas import tpu as pltpu
from jax.experimental.pallas import tpu_sc as plsc

_N = 1024
_IN = 128
_HID = 64
_E = 65536
_MAXN = 15

_EPT = _E // 16
_ROWS = 32
_COLS = _EPT // _ROWS
_HALF = _N * _N // 2
_ZCHUNK = _HALF // 16

_ALPHA = 1.6732632423543772
_SCALE = 1.0507009873554805


def _selu(v):
    return _SCALE * jnp.where(v > 0, v, _ALPHA * (jnp.exp(v) - 1.0))


def _sc_adj_body(edge_hbm, zeros_hbm, out_hbm, src_v, dst_v, idx2d, val2d,
                 m_spmem, zsem, esem):
    c = lax.axis_index("c")
    s = lax.axis_index("s")
    base = s * _EPT

    zcp = pltpu.make_async_copy(
        zeros_hbm, m_spmem.at[pl.ds(s * _ZCHUNK, _ZCHUNK)], zsem)
    zcp.start()
    e0 = pltpu.make_async_copy(edge_hbm.at[0, pl.ds(base, _EPT)], src_v, esem)
    e1 = pltpu.make_async_copy(edge_hbm.at[1, pl.ds(base, _EPT)], dst_v, esem)
    e0.start()
    e1.start()
    e0.wait()
    e1.wait()

    ones = jnp.full((16,), 1.0, jnp.float32)
    zeros16 = jnp.zeros((16,), jnp.float32)
    for j in range(_ROWS):
        def body(i, carry, j=j):
            t = j * _COLS + i * 16
            src = src_v[pl.ds(t, 16)]
            dst = dst_v[pl.ds(t, 16)]
            own = (src >> 9) == c
            f = (src & ~127) * _N + dst * 128 + (src & 127) - c * _HALF
            idx2d[j, pl.ds(i * 16, 16)] = jnp.where(own, f, 0)
            val2d[j, pl.ds(i * 16, 16)] = jnp.where(own, ones, zeros16)
            return carry
        lax.fori_loop(0, _COLS // 16, body, 0)

    zcp.wait()
    plsc.subcore_barrier()
    for j in range(_ROWS):
        pltpu.sync_copy(val2d.at[j], m_spmem.at[idx2d.at[j]], add=True)
    plsc.subcore_barrier()
    pltpu.sync_copy(m_spmem.at[pl.ds(s * _ZCHUNK, _ZCHUNK)],
                    out_hbm.at[pl.ds(c * _HALF + s * _ZCHUNK, _ZCHUNK)])


@functools.cache
def _sc_build_adj():
    return pl.kernel(
        _sc_adj_body,
        out_type=jax.ShapeDtypeStruct((_N * _N,), jnp.float32),
        mesh=plsc.VectorSubcoreMesh(core_axis_name="c", subcore_axis_name="s"),
        scratch_types=[
            pltpu.VMEM((_EPT,), jnp.int32),
            pltpu.VMEM((_EPT,), jnp.int32),
            pltpu.VMEM((_ROWS, _COLS), jnp.int32),
            pltpu.VMEM((_ROWS, _COLS), jnp.float32),
            pltpu.VMEM_SHARED((_HALF,), jnp.float32),
            pltpu.SemaphoreType.DMA,
            pltpu.SemaphoreType.DMA,
        ],
    )


def _gcn_body(m_ref, x_ref, w1_ref, b1_ref, w2_ref, b2_ref, h2_ref):
    deg = jnp.zeros((_N, 1), jnp.float32) + 1.0
    for k in range(8):
        deg = deg + jnp.sum(m_ref[k], axis=1, keepdims=True)
    dinv = lax.rsqrt(deg)
    d2 = dinv * dinv

    def agg(u):
        y = jnp.dot(m_ref[0], u[0:128],
                    preferred_element_type=jnp.float32)
        for k in range(1, 8):
            y = y + jnp.dot(m_ref[k], u[128 * k:128 * (k + 1)],
                            preferred_element_type=jnp.float32)
        return y

    z = jnp.dot(x_ref[...], w1_ref[...], preferred_element_type=jnp.float32)
    h = _selu(dinv * agg(dinv * z) + d2 * z + b1_ref[...])

    z = jnp.dot(h, w2_ref[...], preferred_element_type=jnp.float32)
    h2_ref[...] = _selu(dinv * agg(dinv * z) + d2 * z + b2_ref[...])


_BK = 4096
_KSTEPS = _E // _BK


def _fc_body(h_ref, w_ref, b1_ref, w2_ref, b2_ref, o_ref, acc_ref):
    k = pl.program_id(0)

    @pl.when(k == 0)
    def _():
        acc_ref[...] = b1_ref[...]

    acc_ref[...] += jnp.dot(h_ref[...], w_ref[...],
                            preferred_element_type=jnp.float32)

    @pl.when(k == _KSTEPS - 1)
    def _():
        a = _selu(acc_ref[...])
        o_ref[...] = jnp.dot(a, w2_ref[...],
                             preferred_element_type=jnp.float32) + b2_ref[...]


def kernel(x, edge_index, W1, b1, W2, b2, fc1_w, fc1_b, fc2_w, fc2_b):
    ei = edge_index.astype(jnp.int32)
    zeros = jnp.zeros((_ZCHUNK,), jnp.float32)

    m = _sc_build_adj()(ei, zeros).reshape(8, _N, 128)

    h2 = pl.pallas_call(
        _gcn_body,
        out_shape=jax.ShapeDtypeStruct((_N, _HID), jnp.float32),
    )(m, x, W1, b1.reshape(1, _HID), W2, b2.reshape(1, _HID))

    hflat = h2.reshape(1, _E)
    out = pl.pallas_call(
        _fc_body,
        grid=(_KSTEPS,),
        in_specs=[
            pl.BlockSpec((1, _BK), lambda k: (0, k)),
            pl.BlockSpec((_BK, _N), lambda k: (k, 0)),
            pl.BlockSpec((1, _N), lambda k: (0, 0)),
            pl.BlockSpec((_N, _MAXN), lambda k: (0, 0)),
            pl.BlockSpec((1, _MAXN), lambda k: (0, 0)),
        ],
        out_specs=pl.BlockSpec((1, _MAXN), lambda k: (0, 0)),
        out_shape=jax.ShapeDtypeStruct((1, _MAXN), jnp.float32),
        scratch_shapes=[pltpu.VMEM((1, _N), jnp.float32)],
    )(hflat, fc1_w, fc1_b.reshape(1, _N), fc2_w, fc2_b.reshape(1, _MAXN))
    return out


# speedup vs baseline: 1.2693x; 1.2693x over previous
import functools

import jax
import jax.numpy as jnp
from jax import lax
from jax.experimental import pallas as pl
from jax.experimental.pallas import tpu as pltpu
from jax.experimental.pallas import tpu_sc as plsc

_N = 1024
_IN = 128
_HID = 64
_E = 65536
_MAXN = 15

_EPT = _E // 16
_ROWS = 32
_COLS = _EPT // _ROWS
_HALF = _N * _N // 2
_ZCHUNK = _HALF // 16

_ALPHA = 1.6732632423543772
_SCALE = 1.0507009873554805


def _selu(v):
    return _SCALE * jnp.where(v > 0, v, _ALPHA * (jnp.exp(v) - 1.0))


def _sc_adj_body(edge_hbm, zeros_hbm, out_hbm, src_v, dst_v, idx2d, val2d,
                 m_spmem, zsem, esem):
    c = lax.axis_index("c")
    s = lax.axis_index("s")
    base = s * _EPT

    zcp = pltpu.make_async_copy(
        zeros_hbm, m_spmem.at[pl.ds(s * _ZCHUNK, _ZCHUNK)], zsem)
    zcp.start()
    e0 = pltpu.make_async_copy(edge_hbm.at[0, pl.ds(base, _EPT)], src_v, esem)
    e1 = pltpu.make_async_copy(edge_hbm.at[1, pl.ds(base, _EPT)], dst_v, esem)
    e0.start()
    e1.start()
    e0.wait()
    e1.wait()

    ones = jnp.full((16,), 1.0, jnp.float32)
    zeros16 = jnp.zeros((16,), jnp.float32)
    for j in range(_ROWS):
        def body(i, carry, j=j):
            t = j * _COLS + i * 16
            src = src_v[pl.ds(t, 16)]
            dst = dst_v[pl.ds(t, 16)]
            own = (src >> 9) == c
            f = (src & ~127) * _N + dst * 128 + (src & 127) - c * _HALF
            idx2d[j, pl.ds(i * 16, 16)] = jnp.where(own, f, f & (_HALF - 1))
            val2d[j, pl.ds(i * 16, 16)] = jnp.where(own, ones, zeros16)
            return carry
        lax.fori_loop(0, _COLS // 16, body, 0)

    zcp.wait()
    plsc.subcore_barrier()
    for j in range(_ROWS):
        pltpu.sync_copy(val2d.at[j], m_spmem.at[idx2d.at[j]], add=True)
    plsc.subcore_barrier()
    pltpu.sync_copy(m_spmem.at[pl.ds(s * _ZCHUNK, _ZCHUNK)],
                    out_hbm.at[pl.ds(c * _HALF + s * _ZCHUNK, _ZCHUNK)])


@functools.cache
def _sc_build_adj():
    return pl.kernel(
        _sc_adj_body,
        out_type=jax.ShapeDtypeStruct((_N * _N,), jnp.float32),
        mesh=plsc.VectorSubcoreMesh(core_axis_name="c", subcore_axis_name="s"),
        scratch_types=[
            pltpu.VMEM((_EPT,), jnp.int32),
            pltpu.VMEM((_EPT,), jnp.int32),
            pltpu.VMEM((_ROWS, _COLS), jnp.int32),
            pltpu.VMEM((_ROWS, _COLS), jnp.float32),
            pltpu.VMEM_SHARED((_HALF,), jnp.float32),
            pltpu.SemaphoreType.DMA,
            pltpu.SemaphoreType.DMA,
        ],
    )


def _gcn_body(m_ref, x_ref, w1_ref, b1_ref, w2_ref, b2_ref, h2_ref):
    deg = jnp.zeros((_N, 1), jnp.float32) + 1.0
    for k in range(8):
        deg = deg + jnp.sum(m_ref[k], axis=1, keepdims=True)
    dinv = lax.rsqrt(deg)
    d2 = dinv * dinv

    def agg(u):
        y = jnp.dot(m_ref[0], u[0:128],
                    preferred_element_type=jnp.float32)
        for k in range(1, 8):
            y = y + jnp.dot(m_ref[k], u[128 * k:128 * (k + 1)],
                            preferred_element_type=jnp.float32)
        return y

    z = jnp.dot(x_ref[...], w1_ref[...], preferred_element_type=jnp.float32)
    h = _selu(dinv * agg(dinv * z) + d2 * z + b1_ref[...])

    z = jnp.dot(h, w2_ref[...], preferred_element_type=jnp.float32)
    h2_ref[...] = _selu(dinv * agg(dinv * z) + d2 * z + b2_ref[...])


_BK = 4096
_KSTEPS = _E // _BK


def _fc_body(h_ref, w_ref, b1_ref, w2_ref, b2_ref, o_ref, acc_ref):
    k = pl.program_id(0)

    @pl.when(k == 0)
    def _():
        acc_ref[...] = b1_ref[...]

    acc_ref[...] += jnp.dot(h_ref[...], w_ref[...],
                            preferred_element_type=jnp.float32)

    @pl.when(k == _KSTEPS - 1)
    def _():
        a = _selu(acc_ref[...])
        o_ref[...] = jnp.dot(a, w2_ref[...],
                             preferred_element_type=jnp.float32) + b2_ref[...]


def kernel(x, edge_index, W1, b1, W2, b2, fc1_w, fc1_b, fc2_w, fc2_b):
    ei = edge_index.astype(jnp.int32)
    zeros = jnp.zeros((_ZCHUNK,), jnp.float32)

    m = _sc_build_adj()(ei, zeros).reshape(8, _N, 128)

    h2 = pl.pallas_call(
        _gcn_body,
        out_shape=jax.ShapeDtypeStruct((_N, _HID), jnp.float32),
    )(m, x, W1, b1.reshape(1, _HID), W2, b2.reshape(1, _HID))

    hflat = h2.reshape(1, _E)
    out = pl.pallas_call(
        _fc_body,
        grid=(_KSTEPS,),
        in_specs=[
            pl.BlockSpec((1, _BK), lambda k: (0, k)),
            pl.BlockSpec((_BK, _N), lambda k: (k, 0)),
            pl.BlockSpec((1, _N), lambda k: (0, 0)),
            pl.BlockSpec((_N, _MAXN), lambda k: (0, 0)),
            pl.BlockSpec((1, _MAXN), lambda k: (0, 0)),
        ],
        out_specs=pl.BlockSpec((1, _MAXN), lambda k: (0, 0)),
        out_shape=jax.ShapeDtypeStruct((1, _MAXN), jnp.float32),
        scratch_shapes=[pltpu.VMEM((1, _N), jnp.float32)],
    )(hflat, fc1_w, fc1_b.reshape(1, _N), fc2_w, fc2_b.reshape(1, _MAXN))
    return out


# speedup vs baseline: 1.2716x; 1.0018x over previous
import functools

import jax
import jax.numpy as jnp
from jax import lax
from jax.experimental import pallas as pl
from jax.experimental.pallas import tpu as pltpu
from jax.experimental.pallas import tpu_sc as plsc

_N = 1024
_IN = 128
_HID = 64
_E = 65536
_MAXN = 15

_EPT = _E // 16
_ROWS = 32
_COLS = _EPT // _ROWS
_HALF = _N * _N // 2
_ZCHUNK = _HALF // 16

_ALPHA = 1.6732632423543772
_SCALE = 1.0507009873554805


def _selu(v):
    return _SCALE * jnp.where(v > 0, v, _ALPHA * (jnp.exp(v) - 1.0))


def _sc_adj_body(edge_hbm, zeros_hbm, out_hbm, src_v, dst_v, idx2d, val2d,
                 m_spmem, zsem, esem):
    c = lax.axis_index("c")
    s = lax.axis_index("s")
    base = s * _EPT

    zcp = pltpu.make_async_copy(
        zeros_hbm, m_spmem.at[pl.ds(s * _ZCHUNK, _ZCHUNK)], zsem)
    zcp.start()
    e0 = pltpu.make_async_copy(edge_hbm.at[0, pl.ds(base, _EPT)], src_v, esem)
    e1 = pltpu.make_async_copy(edge_hbm.at[1, pl.ds(base, _EPT)], dst_v, esem)
    e0.start()
    e1.start()
    e0.wait()
    e1.wait()

    ones = jnp.full((16,), 1.0, jnp.float32)
    zeros16 = jnp.zeros((16,), jnp.float32)
    for j in range(_ROWS):
        def body(i, carry, j=j):
            t = j * _COLS + i * 16
            src = src_v[pl.ds(t, 16)]
            dst = dst_v[pl.ds(t, 16)]
            own = (src >> 9) == c
            f = (src & ~127) * _N + dst * 128 + (src & 127) - c * _HALF
            idx2d[j, pl.ds(i * 16, 16)] = jnp.where(own, f, f & (_HALF - 1))
            val2d[j, pl.ds(i * 16, 16)] = jnp.where(own, ones, zeros16)
            return carry
        lax.fori_loop(0, _COLS // 16, body, 0)

    zcp.wait()
    plsc.subcore_barrier()
    for j in range(_ROWS):
        pltpu.sync_copy(val2d.at[j], m_spmem.at[idx2d.at[j]], add=True)
    plsc.subcore_barrier()
    pltpu.sync_copy(m_spmem.at[pl.ds(s * _ZCHUNK, _ZCHUNK)],
                    out_hbm.at[pl.ds(c * _HALF + s * _ZCHUNK, _ZCHUNK)])


@functools.cache
def _sc_build_adj():
    return pl.kernel(
        _sc_adj_body,
        out_type=jax.ShapeDtypeStruct((_N * _N,), jnp.float32),
        mesh=plsc.VectorSubcoreMesh(core_axis_name="c", subcore_axis_name="s"),
        scratch_types=[
            pltpu.VMEM((_EPT,), jnp.int32),
            pltpu.VMEM((_EPT,), jnp.int32),
            pltpu.VMEM((_ROWS, _COLS), jnp.int32),
            pltpu.VMEM((_ROWS, _COLS), jnp.float32),
            pltpu.VMEM_SHARED((_HALF,), jnp.float32),
            pltpu.SemaphoreType.DMA,
            pltpu.SemaphoreType.DMA,
        ],
    )


def _gcn_body(m_ref, x_ref, w1_ref, b1_ref, w2_ref, b2_ref, h2_ref):
    deg = jnp.zeros((_N, 1), jnp.float32) + 1.0
    for k in range(8):
        deg = deg + jnp.sum(m_ref[k], axis=1, keepdims=True)
    dinv = lax.rsqrt(deg)
    d2 = dinv * dinv

    def agg(u):
        y = jnp.dot(m_ref[0], u[0:128],
                    preferred_element_type=jnp.float32)
        for k in range(1, 8):
            y = y + jnp.dot(m_ref[k], u[128 * k:128 * (k + 1)],
                            preferred_element_type=jnp.float32)
        return y

    z = jnp.dot(x_ref[...], w1_ref[...], preferred_element_type=jnp.float32)
    h = _selu(dinv * agg(dinv * z) + d2 * z + b1_ref[...])

    z = jnp.dot(h, w2_ref[...], preferred_element_type=jnp.float32)
    h2_ref[...] = _selu(dinv * agg(dinv * z) + d2 * z + b2_ref[...])


_BK = 2048
_KSTEPS = _E // (2 * _BK)


def _fc_body(ha_ref, hb_ref, wa_ref, wb_ref, b1_ref, w2_ref, b2_ref, o_ref,
             acc_ref):
    k = pl.program_id(0)

    @pl.when(k == 0)
    def _():
        acc_ref[...] = b1_ref[...]

    acc_ref[...] += (
        jnp.dot(ha_ref[...], wa_ref[...], preferred_element_type=jnp.float32)
        + jnp.dot(hb_ref[...], wb_ref[...], preferred_element_type=jnp.float32))

    @pl.when(k == _KSTEPS - 1)
    def _():
        a = _selu(acc_ref[...])
        o_ref[...] = jnp.dot(a, w2_ref[...],
                             preferred_element_type=jnp.float32) + b2_ref[...]


def kernel(x, edge_index, W1, b1, W2, b2, fc1_w, fc1_b, fc2_w, fc2_b):
    ei = edge_index.astype(jnp.int32)
    zeros = jnp.zeros((_ZCHUNK,), jnp.float32)

    m = _sc_build_adj()(ei, zeros).reshape(8, _N, 128)

    h2 = pl.pallas_call(
        _gcn_body,
        out_shape=jax.ShapeDtypeStruct((_N, _HID), jnp.float32),
    )(m, x, W1, b1.reshape(1, _HID), W2, b2.reshape(1, _HID))

    hflat = h2.reshape(1, _E)
    out = pl.pallas_call(
        _fc_body,
        grid=(_KSTEPS,),
        in_specs=[
            pl.BlockSpec((1, _BK), lambda k: (0, 2 * k)),
            pl.BlockSpec((1, _BK), lambda k: (0, 2 * k + 1)),
            pl.BlockSpec((_BK, _N), lambda k: (2 * k, 0)),
            pl.BlockSpec((_BK, _N), lambda k: (2 * k + 1, 0)),
            pl.BlockSpec((1, _N), lambda k: (0, 0)),
            pl.BlockSpec((_N, _MAXN), lambda k: (0, 0)),
            pl.BlockSpec((1, _MAXN), lambda k: (0, 0)),
        ],
        out_specs=pl.BlockSpec((1, _MAXN), lambda k: (0, 0)),
        out_shape=jax.ShapeDtypeStruct((1, _MAXN), jnp.float32),
        scratch_shapes=[pltpu.VMEM((1, _N), jnp.float32)],
    )(hflat, hflat, fc1_w, fc1_w, fc1_b.reshape(1, _N), fc2_w,
      fc2_b.reshape(1, _MAXN))
    return out


# speedup vs baseline: 1.2948x; 1.0183x over previous
import functools

import jax
import jax.numpy as jnp
from jax import lax
from jax.experimental import pallas as pl
from jax.experimental.pallas import tpu as pltpu
from jax.experimental.pallas import tpu_sc as plsc

_N = 1024
_IN = 128
_HID = 64
_E = 65536
_MAXN = 15

_EPT = _E // 16
_ROWS = 32
_COLS = _EPT // _ROWS
_HALF = _N * _N // 2
_ZCHUNK = _HALF // 16

_ALPHA = 1.6732632423543772
_SCALE = 1.0507009873554805


def _selu(v):
    return _SCALE * jnp.where(v > 0, v, _ALPHA * (jnp.exp(v) - 1.0))


def _sc_adj_body(edge_hbm, zeros_hbm, out_hbm, src_v, dst_v, idx2d, val2d,
                 m_spmem, zsem, esem, ssem):
    c = lax.axis_index("c")
    s = lax.axis_index("s")
    base = s * _EPT

    zcp = pltpu.make_async_copy(
        zeros_hbm, m_spmem.at[pl.ds(s * _ZCHUNK, _ZCHUNK)], zsem)
    zcp.start()
    e0 = pltpu.make_async_copy(edge_hbm.at[0, pl.ds(base, _EPT)], src_v, esem)
    e1 = pltpu.make_async_copy(edge_hbm.at[1, pl.ds(base, _EPT)], dst_v, esem)
    e0.start()
    e1.start()
    e0.wait()
    e1.wait()

    ones = jnp.full((16,), 1.0, jnp.float32)
    zeros16 = jnp.zeros((16,), jnp.float32)
    for j in range(_ROWS):
        def body(i, carry, j=j):
            t = j * _COLS + i * 16
            src = src_v[pl.ds(t, 16)]
            dst = dst_v[pl.ds(t, 16)]
            own = (src >> 9) == c
            f = (src & ~127) * _N + dst * 128 + (src & 127) - c * _HALF
            idx2d[j, pl.ds(i * 16, 16)] = jnp.where(own, f, f & (_HALF - 1))
            val2d[j, pl.ds(i * 16, 16)] = jnp.where(own, ones, zeros16)
            return carry
        lax.fori_loop(0, _COLS // 16, body, 0)

    zcp.wait()
    plsc.subcore_barrier()
    cps = [pltpu.async_copy(val2d.at[j], m_spmem.at[idx2d.at[j]], ssem,
                            add=True)
           for j in range(_ROWS)]
    for cp in cps:
        cp.wait()
    plsc.subcore_barrier()
    pltpu.sync_copy(m_spmem.at[pl.ds(s * _ZCHUNK, _ZCHUNK)],
                    out_hbm.at[pl.ds(c * _HALF + s * _ZCHUNK, _ZCHUNK)])


@functools.cache
def _sc_build_adj():
    return pl.kernel(
        _sc_adj_body,
        out_type=jax.ShapeDtypeStruct((_N * _N,), jnp.float32),
        mesh=plsc.VectorSubcoreMesh(core_axis_name="c", subcore_axis_name="s"),
        scratch_types=[
            pltpu.VMEM((_EPT,), jnp.int32),
            pltpu.VMEM((_EPT,), jnp.int32),
            pltpu.VMEM((_ROWS, _COLS), jnp.int32),
            pltpu.VMEM((_ROWS, _COLS), jnp.float32),
            pltpu.VMEM_SHARED((_HALF,), jnp.float32),
            pltpu.SemaphoreType.DMA,
            pltpu.SemaphoreType.DMA,
            pltpu.SemaphoreType.DMA,
        ],
    )


def _gcn_body(m_ref, x_ref, w1_ref, b1_ref, w2_ref, b2_ref, h2_ref):
    deg = jnp.zeros((_N, 1), jnp.float32) + 1.0
    for k in range(8):
        deg = deg + jnp.sum(m_ref[k], axis=1, keepdims=True)
    dinv = lax.rsqrt(deg)
    d2 = dinv * dinv

    def agg(u):
        y = jnp.dot(m_ref[0], u[0:128],
                    preferred_element_type=jnp.float32)
        for k in range(1, 8):
            y = y + jnp.dot(m_ref[k], u[128 * k:128 * (k + 1)],
                            preferred_element_type=jnp.float32)
        return y

    z = jnp.dot(x_ref[...], w1_ref[...], preferred_element_type=jnp.float32)
    h = _selu(dinv * agg(dinv * z) + d2 * z + b1_ref[...])

    z = jnp.dot(h, w2_ref[...], preferred_element_type=jnp.float32)
    h2_ref[...] = _selu(dinv * agg(dinv * z) + d2 * z + b2_ref[...])


_BK = 2048
_KSTEPS = _E // (2 * _BK)


def _fc_body(ha_ref, hb_ref, wa_ref, wb_ref, b1_ref, w2_ref, b2_ref, o_ref,
             acc_ref):
    k = pl.program_id(0)

    @pl.when(k == 0)
    def _():
        acc_ref[...] = b1_ref[...]

    acc_ref[...] += (
        jnp.dot(ha_ref[...], wa_ref[...], preferred_element_type=jnp.float32)
        + jnp.dot(hb_ref[...], wb_ref[...], preferred_element_type=jnp.float32))

    @pl.when(k == _KSTEPS - 1)
    def _():
        a = _selu(acc_ref[...])
        o_ref[...] = jnp.dot(a, w2_ref[...],
                             preferred_element_type=jnp.float32) + b2_ref[...]


def kernel(x, edge_index, W1, b1, W2, b2, fc1_w, fc1_b, fc2_w, fc2_b):
    ei = edge_index.astype(jnp.int32)
    zeros = jnp.zeros((_ZCHUNK,), jnp.float32)

    m = _sc_build_adj()(ei, zeros).reshape(8, _N, 128)

    h2 = pl.pallas_call(
        _gcn_body,
        out_shape=jax.ShapeDtypeStruct((_N, _HID), jnp.float32),
    )(m, x, W1, b1.reshape(1, _HID), W2, b2.reshape(1, _HID))

    hflat = h2.reshape(1, _E)
    out = pl.pallas_call(
        _fc_body,
        grid=(_KSTEPS,),
        in_specs=[
            pl.BlockSpec((1, _BK), lambda k: (0, 2 * k)),
            pl.BlockSpec((1, _BK), lambda k: (0, 2 * k + 1)),
            pl.BlockSpec((_BK, _N), lambda k: (2 * k, 0)),
            pl.BlockSpec((_BK, _N), lambda k: (2 * k + 1, 0)),
            pl.BlockSpec((1, _N), lambda k: (0, 0)),
            pl.BlockSpec((_N, _MAXN), lambda k: (0, 0)),
            pl.BlockSpec((1, _MAXN), lambda k: (0, 0)),
        ],
        out_specs=pl.BlockSpec((1, _MAXN), lambda k: (0, 0)),
        out_shape=jax.ShapeDtypeStruct((1, _MAXN), jnp.float32),
        scratch_shapes=[pltpu.VMEM((1, _N), jnp.float32)],
    )(hflat, hflat, fc1_w, fc1_w, fc1_b.reshape(1, _N), fc2_w,
      fc2_b.reshape(1, _MAXN))
    return out


# speedup vs baseline: 1.3078x; 1.0101x over previous
import functools

import jax
import jax.numpy as jnp
from jax import lax
from jax.experimental import pallas as pl
from jax.experimental.pallas import tpu as pltpu
from jax.experimental.pallas import tpu_sc as plsc

_N = 1024
_IN = 128
_HID = 64
_E = 65536
_MAXN = 15

_EPT = _E // 16
_ROWS = 32
_COLS = _EPT // _ROWS
_HALF = _N * _N // 2
_ZCHUNK = _HALF // 16

_ALPHA = 1.6732632423543772
_SCALE = 1.0507009873554805


def _selu(v):
    return _SCALE * jnp.where(v > 0, v, _ALPHA * (jnp.exp(v) - 1.0))


def _sc_adj_body(edge_hbm, zeros_hbm, out_hbm, src_v, dst_v, idx2d, val2d,
                 m_spmem, zsem, esem, ssem):
    c = lax.axis_index("c")
    s = lax.axis_index("s")
    base = s * _EPT

    zcp = pltpu.make_async_copy(
        zeros_hbm, m_spmem.at[pl.ds(s * _ZCHUNK, _ZCHUNK)], zsem)
    zcp.start()
    e0 = pltpu.make_async_copy(edge_hbm.at[0, pl.ds(base, _EPT)], src_v, esem)
    e1 = pltpu.make_async_copy(edge_hbm.at[1, pl.ds(base, _EPT)], dst_v, esem)
    e0.start()
    e1.start()
    e0.wait()
    e1.wait()

    ones = jnp.full((16,), 1.0, jnp.float32)
    zeros16 = jnp.zeros((16,), jnp.float32)
    for j in range(_ROWS):
        def body(i, carry, j=j):
            t = j * _COLS + i * 16
            src = src_v[pl.ds(t, 16)]
            dst = dst_v[pl.ds(t, 16)]
            own = (src >> 9) == c
            f = (src & ~127) * _N + dst * 128 + (src & 127) - c * _HALF
            idx2d[j, pl.ds(i * 16, 16)] = jnp.where(own, f, f & (_HALF - 1))
            val2d[j, pl.ds(i * 16, 16)] = jnp.where(own, ones, zeros16)
            return carry
        lax.fori_loop(0, _COLS // 16, body, 0)

    zcp.wait()
    plsc.subcore_barrier()
    cps = [pltpu.async_copy(val2d.at[j], m_spmem.at[idx2d.at[j]], ssem,
                            add=True)
           for j in range(_ROWS)]
    for cp in cps:
        cp.wait()
    plsc.subcore_barrier()
    pltpu.sync_copy(m_spmem.at[pl.ds(s * _ZCHUNK, _ZCHUNK)],
                    out_hbm.at[pl.ds(c * _HALF + s * _ZCHUNK, _ZCHUNK)])


@functools.cache
def _sc_build_adj():
    return pl.kernel(
        _sc_adj_body,
        out_type=jax.ShapeDtypeStruct((_N * _N,), jnp.float32),
        mesh=plsc.VectorSubcoreMesh(core_axis_name="c", subcore_axis_name="s"),
        scratch_types=[
            pltpu.VMEM((_EPT,), jnp.int32),
            pltpu.VMEM((_EPT,), jnp.int32),
            pltpu.VMEM((_ROWS, _COLS), jnp.int32),
            pltpu.VMEM((_ROWS, _COLS), jnp.float32),
            pltpu.VMEM_SHARED((_HALF,), jnp.float32),
            pltpu.SemaphoreType.DMA,
            pltpu.SemaphoreType.DMA,
            pltpu.SemaphoreType.DMA,
        ],
    )


def _gcn_body(m_ref, x_ref, w1_ref, b1_ref, w2_ref, b2_ref, h2_ref):
    deg = jnp.zeros((_N, 1), jnp.float32) + 1.0
    for k in range(8):
        deg = deg + jnp.sum(m_ref[k], axis=1, keepdims=True)
    dinv = lax.rsqrt(deg)
    d2 = dinv * dinv

    def agg(u):
        y = jnp.dot(m_ref[0], u[0:128],
                    preferred_element_type=jnp.float32)
        for k in range(1, 8):
            y = y + jnp.dot(m_ref[k], u[128 * k:128 * (k + 1)],
                            preferred_element_type=jnp.float32)
        return y

    z = jnp.dot(x_ref[...], w1_ref[...], preferred_element_type=jnp.float32)
    h = _selu(dinv * agg(dinv * z) + d2 * z + b1_ref[...])

    z = jnp.dot(h, w2_ref[...], preferred_element_type=jnp.float32)
    h2_ref[...] = _selu(dinv * agg(dinv * z) + d2 * z + b2_ref[...])


_BKN = 64
_KSTEPS = _N // _BKN


def _fc_body(h_ref, w_ref, b1_ref, w2_ref, b2_ref, o_ref, acc_ref):
    k = pl.program_id(0)

    @pl.when(k == 0)
    def _():
        acc_ref[...] = b1_ref[...]

    w3 = w_ref[...].reshape(_BKN, _HID, _N)
    part = lax.dot_general(h_ref[...], w3,
                           dimension_numbers=(((1,), (1,)), ((0,), (0,))),
                           preferred_element_type=jnp.float32)
    acc_ref[...] += jnp.sum(part, axis=0, keepdims=True)

    @pl.when(k == _KSTEPS - 1)
    def _():
        a = _selu(acc_ref[...])
        o_ref[...] = jnp.dot(a, w2_ref[...],
                             preferred_element_type=jnp.float32) + b2_ref[...]


def kernel(x, edge_index, W1, b1, W2, b2, fc1_w, fc1_b, fc2_w, fc2_b):
    ei = edge_index.astype(jnp.int32)
    zeros = jnp.zeros((_ZCHUNK,), jnp.float32)

    m = _sc_build_adj()(ei, zeros).reshape(8, _N, 128)

    h2 = pl.pallas_call(
        _gcn_body,
        out_shape=jax.ShapeDtypeStruct((_N, _HID), jnp.float32),
    )(m, x, W1, b1.reshape(1, _HID), W2, b2.reshape(1, _HID))

    out = pl.pallas_call(
        _fc_body,
        grid=(_KSTEPS,),
        in_specs=[
            pl.BlockSpec((_BKN, _HID), lambda k: (k, 0)),
            pl.BlockSpec((_BKN * _HID, _N), lambda k: (k, 0)),
            pl.BlockSpec((1, _N), lambda k: (0, 0)),
            pl.BlockSpec((_N, _MAXN), lambda k: (0, 0)),
            pl.BlockSpec((1, _MAXN), lambda k: (0, 0)),
        ],
        out_specs=pl.BlockSpec((1, _MAXN), lambda k: (0, 0)),
        out_shape=jax.ShapeDtypeStruct((1, _MAXN), jnp.float32),
        scratch_shapes=[pltpu.VMEM((1, _N), jnp.float32)],
    )(h2, fc1_w, fc1_b.reshape(1, _N), fc2_w, fc2_b.reshape(1, _MAXN))
    return out


# speedup vs baseline: 1.3664x; 1.0448x over previous
import functools

import jax
import jax.numpy as jnp
from jax import lax
from jax.experimental import pallas as pl
from jax.experimental.pallas import tpu as pltpu
from jax.experimental.pallas import tpu_sc as plsc

_N = 1024
_IN = 128
_HID = 64
_E = 65536
_MAXN = 15

_EPT = _E // 16
_ROWS = 32
_COLS = _EPT // _ROWS
_HALF = _N * _N // 2
_ZCHUNK = _HALF // 16

_ALPHA = 1.6732632423543772
_SCALE = 1.0507009873554805


def _selu(v):
    return _SCALE * jnp.where(v > 0, v, _ALPHA * (jnp.exp(v) - 1.0))


def _sc_adj_body(edge_hbm, zeros_hbm, out_hbm, src_v, dst_v, idx2d, val2d,
                 m_spmem, zsem, esem, ssem):
    c = lax.axis_index("c")
    s = lax.axis_index("s")
    base = s * _EPT

    zcp = pltpu.make_async_copy(
        zeros_hbm, m_spmem.at[pl.ds(s * _ZCHUNK, _ZCHUNK)], zsem)
    zcp.start()
    e0 = pltpu.make_async_copy(edge_hbm.at[0, pl.ds(base, _EPT)], src_v, esem)
    e1 = pltpu.make_async_copy(edge_hbm.at[1, pl.ds(base, _EPT)], dst_v, esem)
    e0.start()
    e1.start()
    e0.wait()
    e1.wait()

    ones = jnp.full((16,), 1.0, jnp.float32)
    zeros16 = jnp.zeros((16,), jnp.float32)
    for j in range(_ROWS):
        def body(i, carry, j=j):
            t = j * _COLS + i * 16
            src = src_v[pl.ds(t, 16)]
            dst = dst_v[pl.ds(t, 16)]
            own = (src >> 9) == c
            f = (src & ~127) * _N + dst * 128 + (src & 127) - c * _HALF
            idx2d[j, pl.ds(i * 16, 16)] = jnp.where(own, f, f & (_HALF - 1))
            val2d[j, pl.ds(i * 16, 16)] = jnp.where(own, ones, zeros16)
            return carry
        lax.fori_loop(0, _COLS // 16, body, 0)

    zcp.wait()
    plsc.subcore_barrier()
    cps = [pltpu.async_copy(val2d.at[j], m_spmem.at[idx2d.at[j]], ssem,
                            add=True)
           for j in range(_ROWS)]
    for cp in cps:
        cp.wait()
    plsc.subcore_barrier()
    pltpu.sync_copy(m_spmem.at[pl.ds(s * _ZCHUNK, _ZCHUNK)],
                    out_hbm.at[pl.ds(c * _HALF + s * _ZCHUNK, _ZCHUNK)])


@functools.cache
def _sc_build_adj():
    return pl.kernel(
        _sc_adj_body,
        out_type=jax.ShapeDtypeStruct((_N * _N,), jnp.float32),
        mesh=plsc.VectorSubcoreMesh(core_axis_name="c", subcore_axis_name="s"),
        scratch_types=[
            pltpu.VMEM((_EPT,), jnp.int32),
            pltpu.VMEM((_EPT,), jnp.int32),
            pltpu.VMEM((_ROWS, _COLS), jnp.int32),
            pltpu.VMEM((_ROWS, _COLS), jnp.float32),
            pltpu.VMEM_SHARED((_HALF,), jnp.float32),
            pltpu.SemaphoreType.DMA,
            pltpu.SemaphoreType.DMA,
            pltpu.SemaphoreType.DMA,
        ],
    )


_BKN = 64
_KSTEPS = _N // _BKN


def _gcn_fc_body(m_ref, x_ref, w1_ref, b1_ref, w2_ref, b2_ref, wf_ref,
                 bf1_ref, wf2_ref, bf2_ref, o_ref, h2_ref, acc_ref):
    k = pl.program_id(0)

    @pl.when(k == 0)
    def _():
        deg = jnp.zeros((_N, 1), jnp.float32) + 1.0
        for j in range(8):
            deg = deg + jnp.sum(m_ref[j], axis=1, keepdims=True)
        dinv = lax.rsqrt(deg)
        d2 = dinv * dinv

        def agg(u):
            y = jnp.dot(m_ref[0], u[0:128],
                        preferred_element_type=jnp.float32)
            for j in range(1, 8):
                y = y + jnp.dot(m_ref[j], u[128 * j:128 * (j + 1)],
                                preferred_element_type=jnp.float32)
            return y

        z = jnp.dot(x_ref[...], w1_ref[...],
                    preferred_element_type=jnp.float32)
        h = _selu(dinv * agg(dinv * z) + d2 * z + b1_ref[...])
        z = jnp.dot(h, w2_ref[...], preferred_element_type=jnp.float32)
        h2_ref[...] = _selu(dinv * agg(dinv * z) + d2 * z + b2_ref[...])
        acc_ref[...] = bf1_ref[...]

    h2 = h2_ref[pl.ds(k * _BKN, _BKN), :]
    w3 = wf_ref[...].reshape(_BKN, _HID, _N)
    part = lax.dot_general(h2, w3,
                           dimension_numbers=(((1,), (1,)), ((0,), (0,))),
                           preferred_element_type=jnp.float32)
    acc_ref[...] += jnp.sum(part, axis=0, keepdims=True)

    @pl.when(k == _KSTEPS - 1)
    def _():
        a = _selu(acc_ref[...])
        o_ref[...] = jnp.dot(a, wf2_ref[...],
                             preferred_element_type=jnp.float32) + bf2_ref[...]


def kernel(x, edge_index, W1, b1, W2, b2, fc1_w, fc1_b, fc2_w, fc2_b):
    ei = edge_index.astype(jnp.int32)
    zeros = jnp.zeros((_ZCHUNK,), jnp.float32)

    m = _sc_build_adj()(ei, zeros).reshape(8, _N, 128)

    out = pl.pallas_call(
        _gcn_fc_body,
        grid=(_KSTEPS,),
        in_specs=[
            pl.BlockSpec((8, _N, 128), lambda k: (0, 0, 0)),
            pl.BlockSpec((_N, _IN), lambda k: (0, 0)),
            pl.BlockSpec((_IN, _HID), lambda k: (0, 0)),
            pl.BlockSpec((1, _HID), lambda k: (0, 0)),
            pl.BlockSpec((_HID, _HID), lambda k: (0, 0)),
            pl.BlockSpec((1, _HID), lambda k: (0, 0)),
            pl.BlockSpec((_BKN * _HID, _N), lambda k: (k, 0)),
            pl.BlockSpec((1, _N), lambda k: (0, 0)),
            pl.BlockSpec((_N, _MAXN), lambda k: (0, 0)),
            pl.BlockSpec((1, _MAXN), lambda k: (0, 0)),
        ],
        out_specs=pl.BlockSpec((1, _MAXN), lambda k: (0, 0)),
        out_shape=jax.ShapeDtypeStruct((1, _MAXN), jnp.float32),
        scratch_shapes=[pltpu.VMEM((_N, _HID), jnp.float32),
                        pltpu.VMEM((1, _N), jnp.float32)],
    )(m, x, W1, b1.reshape(1, _HID), W2, b2.reshape(1, _HID),
      fc1_w, fc1_b.reshape(1, _N), fc2_w, fc2_b.reshape(1, _MAXN))
    return out


# speedup vs baseline: 1.3670x; 1.0004x over previous
import functools

import jax
import jax.numpy as jnp
from jax import lax
from jax.experimental import pallas as pl
from jax.experimental.pallas import tpu as pltpu
from jax.experimental.pallas import tpu_sc as plsc

_N = 1024
_IN = 128
_HID = 64
_E = 65536
_MAXN = 15

_EPT = _E // 16
_ROWS = 32
_COLS = _EPT // _ROWS
_HALF = _N * _N // 2
_ZCHUNK = _HALF // 16

_ALPHA = 1.6732632423543772
_SCALE = 1.0507009873554805


def _selu(v):
    return _SCALE * jnp.where(v > 0, v, _ALPHA * (jnp.exp(v) - 1.0))


def _sc_adj_body(edge_hbm, zeros_hbm, out_hbm, src_v, dst_v, idx2d, val2d,
                 m_spmem, zsem, esem, ssem):
    c = lax.axis_index("c")
    s = lax.axis_index("s")
    base = s * _EPT

    zcp = pltpu.make_async_copy(
        zeros_hbm, m_spmem.at[pl.ds(s * _ZCHUNK, _ZCHUNK)], zsem)
    zcp.start()
    e0 = pltpu.make_async_copy(edge_hbm.at[0, pl.ds(base, _EPT)], src_v, esem)
    e1 = pltpu.make_async_copy(edge_hbm.at[1, pl.ds(base, _EPT)], dst_v, esem)
    e0.start()
    e1.start()
    e0.wait()
    e1.wait()

    ones = jnp.full((16,), 1.0, jnp.float32)
    zeros16 = jnp.zeros((16,), jnp.float32)
    for j in range(_ROWS):
        def body(i, carry, j=j):
            t = j * _COLS + i * 16
            src = src_v[pl.ds(t, 16)]
            dst = dst_v[pl.ds(t, 16)]
            own = (src >> 9) == c
            f = (src & ~127) * _N + dst * 128 + (src & 127) - c * _HALF
            idx2d[j, pl.ds(i * 16, 16)] = jnp.where(own, f, f & (_HALF - 1))
            val2d[j, pl.ds(i * 16, 16)] = jnp.where(own, ones, zeros16)
            return carry
        lax.fori_loop(0, _COLS // 16, body, 0)

    zcp.wait()
    plsc.subcore_barrier()
    cps = [pltpu.async_copy(val2d.at[j], m_spmem.at[idx2d.at[j]], ssem,
                            add=True)
           for j in range(_ROWS)]
    for cp in cps:
        cp.wait()
    plsc.subcore_barrier()
    pltpu.sync_copy(m_spmem.at[pl.ds(s * _ZCHUNK, _ZCHUNK)],
                    out_hbm.at[pl.ds(c * _HALF + s * _ZCHUNK, _ZCHUNK)])


@functools.cache
def _sc_build_adj():
    return pl.kernel(
        _sc_adj_body,
        out_type=jax.ShapeDtypeStruct((_N * _N,), jnp.float32),
        mesh=plsc.VectorSubcoreMesh(core_axis_name="c", subcore_axis_name="s"),
        scratch_types=[
            pltpu.VMEM((_EPT,), jnp.int32),
            pltpu.VMEM((_EPT,), jnp.int32),
            pltpu.VMEM((_ROWS, _COLS), jnp.int32),
            pltpu.VMEM((_ROWS, _COLS), jnp.float32),
            pltpu.VMEM_SHARED((_HALF,), jnp.float32),
            pltpu.SemaphoreType.DMA,
            pltpu.SemaphoreType.DMA,
            pltpu.SemaphoreType.DMA,
        ],
    )


_BKN = 32
_KSTEPS = _N // _BKN


def _gcn_fc_body(m_ref, x_ref, w1_ref, b1_ref, w2_ref, b2_ref, wf_ref,
                 bf1_ref, wf2_ref, bf2_ref, o_ref, h2_ref, acc_ref):
    k = pl.program_id(0)

    @pl.when(k == 0)
    def _():
        deg = jnp.zeros((_N, 1), jnp.float32) + 1.0
        for j in range(8):
            deg = deg + jnp.sum(m_ref[j], axis=1, keepdims=True)
        dinv = lax.rsqrt(deg)
        d2 = dinv * dinv

        def agg(u):
            y = jnp.dot(m_ref[0], u[0:128],
                        preferred_element_type=jnp.float32)
            for j in range(1, 8):
                y = y + jnp.dot(m_ref[j], u[128 * j:128 * (j + 1)],
                                preferred_element_type=jnp.float32)
            return y

        z = jnp.dot(x_ref[...], w1_ref[...],
                    preferred_element_type=jnp.float32)
        h = _selu(dinv * agg(dinv * z) + d2 * z + b1_ref[...])
        z = jnp.dot(h, w2_ref[...], preferred_element_type=jnp.float32)
        h2_ref[...] = _selu(dinv * agg(dinv * z) + d2 * z + b2_ref[...])
        acc_ref[...] = bf1_ref[...]

    h2 = h2_ref[pl.ds(k * _BKN, _BKN), :]
    w3 = wf_ref[...].reshape(_BKN, _HID, _N)
    part = lax.dot_general(h2, w3,
                           dimension_numbers=(((1,), (1,)), ((0,), (0,))),
                           preferred_element_type=jnp.float32)
    acc_ref[...] += jnp.sum(part, axis=0, keepdims=True)

    @pl.when(k == _KSTEPS - 1)
    def _():
        a = _selu(acc_ref[...])
        o_ref[...] = jnp.dot(a, wf2_ref[...],
                             preferred_element_type=jnp.float32) + bf2_ref[...]


def kernel(x, edge_index, W1, b1, W2, b2, fc1_w, fc1_b, fc2_w, fc2_b):
    ei = edge_index.astype(jnp.int32)
    zeros = jnp.zeros((_ZCHUNK,), jnp.float32)

    m = _sc_build_adj()(ei, zeros).reshape(8, _N, 128)

    out = pl.pallas_call(
        _gcn_fc_body,
        grid=(_KSTEPS,),
        in_specs=[
            pl.BlockSpec((8, _N, 128), lambda k: (0, 0, 0)),
            pl.BlockSpec((_N, _IN), lambda k: (0, 0)),
            pl.BlockSpec((_IN, _HID), lambda k: (0, 0)),
            pl.BlockSpec((1, _HID), lambda k: (0, 0)),
            pl.BlockSpec((_HID, _HID), lambda k: (0, 0)),
            pl.BlockSpec((1, _HID), lambda k: (0, 0)),
            pl.BlockSpec((_BKN * _HID, _N), lambda k: (k, 0)),
            pl.BlockSpec((1, _N), lambda k: (0, 0)),
            pl.BlockSpec((_N, _MAXN), lambda k: (0, 0)),
            pl.BlockSpec((1, _MAXN), lambda k: (0, 0)),
        ],
        out_specs=pl.BlockSpec((1, _MAXN), lambda k: (0, 0)),
        out_shape=jax.ShapeDtypeStruct((1, _MAXN), jnp.float32),
        scratch_shapes=[pltpu.VMEM((_N, _HID), jnp.float32),
                        pltpu.VMEM((1, _N), jnp.float32)],
    )(m, x, W1, b1.reshape(1, _HID), W2, b2.reshape(1, _HID),
      fc1_w, fc1_b.reshape(1, _N), fc2_w, fc2_b.reshape(1, _MAXN))
    return out
